# baseline (speedup 1.0000x reference)
import functools

import jax
import jax.numpy as jnp
from jax import lax
from jax.experimental import pallas as pl
from jax.experimental.pallas import tpu as pltpu
from jax.experimental.pallas import tpu_sc as plsc

_F32 = jnp.float32
_NC = 2
_NS = 16
_CH = 128


def _full(shape):
    return pl.BlockSpec(shape, lambda i: (0, 0))


def _rows(block_rows, width):
    return pl.BlockSpec((block_rows, width), lambda i: (i, 0))


_PARAMS = pltpu.CompilerParams(dimension_semantics=("arbitrary",))


def _tc_mlp2(x, w1, b1, w2, b2, block_rows, out_rows=None, block_off=0,
             out_dtype=None):
    rows, k = x.shape
    h = w1.shape[1]
    o = w2.shape[1]
    out_rows = rows if out_rows is None else out_rows
    max_blk = rows // block_rows - 1
    out_dtype = _F32 if out_dtype is None else out_dtype

    def body(x_ref, w1_ref, b1_ref, w2_ref, b2_ref, o_ref):
        t = jnp.dot(x_ref[...], w1_ref[...], preferred_element_type=_F32)
        t = jax.nn.gelu(t + b1_ref[...])
        y = jnp.dot(t, w2_ref[...], preferred_element_type=_F32) + b2_ref[...]
        o_ref[...] = y.astype(out_dtype)

    return pl.pallas_call(
        body,
        grid=(out_rows // block_rows,),
        in_specs=[pl.BlockSpec((block_rows, k),
                               lambda i: (jnp.minimum(i + block_off, max_blk), 0)),
                  _full((k, h)), _full((1, h)), _full((h, o)), _full((1, o))],
        out_specs=_rows(block_rows, o),
        out_shape=jax.ShapeDtypeStruct((out_rows, o), out_dtype),
        compiler_params=_PARAMS,
    )(x, w1, b1.reshape(1, -1), w2, b2.reshape(1, -1))


def _tc_embed_node_proj(x, w1, b1, w2, b2, w_sr, block_rows):
    rows, k = x.shape
    d = w2.shape[1]

    def body(x_ref, w1_ref, b1_ref, w2_ref, b2_ref, wsr_ref,
             o_ref, ps_ref, pr_ref):
        t = jnp.dot(x_ref[...], w1_ref[...], preferred_element_type=_F32)
        t = jax.nn.gelu(t + b1_ref[...])
        no = jnp.dot(t, w2_ref[...], preferred_element_type=_F32) + b2_ref[...]
        o_ref[...] = no
        tp = jnp.dot(no, wsr_ref[...], preferred_element_type=_F32)
        ps_ref[...] = tp[:, :d]
        pr_ref[...] = tp[:, d:]

    return pl.pallas_call(
        body,
        grid=(rows // block_rows,),
        in_specs=[_rows(block_rows, k), _full((k, d)), _full((1, d)),
                  _full((d, d)), _full((1, d)), _full((d, 2 * d))],
        out_specs=[_rows(block_rows, d)] * 3,
        out_shape=[jax.ShapeDtypeStruct((rows, d), _F32)] * 3,
        compiler_params=_PARAMS,
    )(x, w1, b1.reshape(1, -1), w2, b2.reshape(1, -1), w_sr)


def _tc_biasrows(g, wg, bcat):
    def body(g_ref, w_ref, b_ref, o_ref):
        o_ref[...] = (jnp.dot(g_ref[...], w_ref[...], preferred_element_type=_F32)
                      + b_ref[...])

    return pl.pallas_call(
        body,
        out_shape=jax.ShapeDtypeStruct(bcat.shape, _F32),
    )(g, wg, bcat)


def _layer_norm(z, g_row, b_row):
    m = jnp.mean(z, axis=-1, keepdims=True)
    v = jnp.mean((z - m) ** 2, axis=-1, keepdims=True)
    return (z - m) / jnp.sqrt(v + 1e-6) * g_row + b_row


def _tc_edge(e, ge, w1e, brow, w2, b2, lng, lnb, block_rows):
    rows, d = e.shape

    def body(e_ref, ge_ref, w1_ref, brow_ref, w2_ref, b2_ref,
             g_ref, b_ref, ne_ref, eo_ref):
        t = jnp.dot(e_ref[...], w1_ref[...], preferred_element_type=_F32)
        t = jax.nn.gelu(t + ge_ref[...] + brow_ref[...])
        ne = jnp.dot(t.astype(jnp.bfloat16), w2_ref[...],
                     preferred_element_type=_F32) + b2_ref[...]
        ne_ref[...] = ne
        z = e_ref[...].astype(_F32) + ne
        eo_ref[...] = _layer_norm(z, g_ref[...], b_ref[...]).astype(jnp.bfloat16)

    return pl.pallas_call(
        body,
        grid=(rows // block_rows,),
        in_specs=[_rows(block_rows, d), _rows(block_rows, d),
                  _full((d, d)), _full((1, d)), _full((d, d)), _full((1, d)),
                  _full((1, d)), _full((1, d))],
        out_specs=[_rows(block_rows, d), _rows(block_rows, d)],
        out_shape=[jax.ShapeDtypeStruct((rows, d), _F32),
                   jax.ShapeDtypeStruct((rows, d), jnp.bfloat16)],
        compiler_params=_PARAMS,
    )(e, ge, w1e.astype(jnp.bfloat16), brow, w2.astype(jnp.bfloat16),
      b2.reshape(1, -1), lng, lnb)


def _tc_node(n, recvs, wn, wr, brow, w2, b2, lng, lnb, block_rows,
             w_sr=None):
    rows, d = n.shape
    nblk = rows // block_rows
    nr = len(recvs)

    def compute(n_ref, r_refs, wn_ref, wr_ref, brow_ref, w2_ref, b2_ref,
                g_ref, b_ref):
        r = r_refs[0][...] + r_refs[1][...]
        for rr in r_refs[2:]:
            r = r + rr[...]
        t = (jnp.dot(n_ref[...], wn_ref[...], preferred_element_type=_F32)
             + jnp.dot(r, wr_ref[...], preferred_element_type=_F32))
        t = jax.nn.gelu(t + brow_ref[...])
        nn = jnp.dot(t, w2_ref[...], preferred_element_type=_F32) + b2_ref[...]
        return _layer_norm(n_ref[...] + nn, g_ref[...], b_ref[...])

    lo = pl.BlockSpec((block_rows, d), lambda i: (i, 0))
    hi = pl.BlockSpec((block_rows, d), lambda i: (i + nblk, 0))
    in_specs = ([lo] + [lo, hi] * nr
                + [_full((d, d)), _full((d, d)), _full((1, d)), _full((d, d)),
                   _full((1, d)), _full((1, d)), _full((1, d))])
    args = ([n] + [rv for rv in recvs for _ in range(2)]
            + [wn, wr, brow, w2, b2.reshape(1, -1), lng, lnb])

    if w_sr is None:
        def body(*refs):
            n_ref = refs[0]
            r_refs = refs[1:1 + 2 * nr]
            (wn_ref, wr_ref, brow_ref, w2_ref, b2_ref, g_ref, b_ref,
             no_ref) = refs[1 + 2 * nr:]
            no_ref[...] = compute(n_ref, r_refs, wn_ref, wr_ref, brow_ref,
                                  w2_ref, b2_ref, g_ref, b_ref)

        return pl.pallas_call(
            body,
            grid=(nblk,),
            in_specs=in_specs,
            out_specs=_rows(block_rows, d),
            out_shape=jax.ShapeDtypeStruct((rows, d), _F32),
            compiler_params=_PARAMS,
        )(*args)

    def body3(*refs):
        n_ref = refs[0]
        r_refs = refs[1:1 + 2 * nr]
        (wn_ref, wr_ref, brow_ref, w2_ref, b2_ref, g_ref, b_ref, wsr_ref,
         no_ref, ps_ref, pr_ref) = refs[1 + 2 * nr:]
        no = compute(n_ref, r_refs, wn_ref, wr_ref, brow_ref,
                     w2_ref, b2_ref, g_ref, b_ref)
        no_ref[...] = no
        tp = jnp.dot(no, wsr_ref[...], preferred_element_type=_F32)
        ps_ref[...] = tp[:, :d]
        pr_ref[...] = tp[:, d:]

    return pl.pallas_call(
        body3,
        grid=(nblk,),
        in_specs=in_specs + [_full((d, 2 * d))],
        out_specs=[_rows(block_rows, d)] * 3,
        out_shape=[jax.ShapeDtypeStruct((rows, d), _F32)] * 3,
        compiler_params=_PARAMS,
    )(*args, w_sr)


def _sc_mesh():
    return plsc.VectorSubcoreMesh(core_axis_name="c", subcore_axis_name="s")


def _sc_gather(ts, tr, sidx, ridx, ep):
    d = ts.shape[1]
    nv = d // 16
    nw = _NC * _NS
    per_w = ep // nw
    ch = _CH // 2
    n_ch = per_w // ch
    nbuf = 4

    @functools.partial(
        pl.kernel,
        out_type=jax.ShapeDtypeStruct((ep, d), _F32),
        mesh=_sc_mesh(),
        scratch_types=(
            [pltpu.VMEM((n_ch, ch), jnp.int32)] * 2
            + [pltpu.VMEM((ch, d), _F32)] * (3 * nbuf)
            + [pltpu.SemaphoreType.DMA] * (2 * nbuf)
        ),
    )
    def k(ts_h, tr_h, si_h, ri_h, o_h, iv_s, iv_r, *bufs):
        bs = bufs[0:nbuf]
        br = bufs[nbuf:2 * nbuf]
        wb = bufs[2 * nbuf:3 * nbuf]
        sem = bufs[3 * nbuf:4 * nbuf]
        wsem = bufs[4 * nbuf:5 * nbuf]
        wid = lax.axis_index("s") * _NC + lax.axis_index("c")
        base = wid * per_w
        pltpu.sync_copy(si_h.at[wid], iv_s)
        pltpu.sync_copy(ri_h.at[wid], iv_r)

        def fire(j, i):
            pltpu.async_copy(ts_h.at[iv_s.at[j]], bs[i], sem[i])
            pltpu.async_copy(tr_h.at[iv_r.at[j]], br[i], sem[i])

        def drain(j, i):
            pltpu.make_async_copy(ts_h.at[iv_s.at[j]], bs[i], sem[i]).wait()
            pltpu.make_async_copy(tr_h.at[iv_r.at[j]], br[i], sem[i]).wait()

            @pl.when(j >= nbuf)
            def _():
                pltpu.make_async_copy(
                    wb[i], o_h.at[pl.ds(base + (j - nbuf) * ch, ch)],
                    wsem[i]).wait()

            @pl.loop(0, ch, unroll=4)
            def _(r):
                for c in range(nv):
                    sl = pl.ds(c * 16, 16)
                    wb[i][r, sl] = bs[i][r, sl] + br[i][r, sl]

            pltpu.async_copy(wb[i], o_h.at[pl.ds(base + j * ch, ch)], wsem[i])

        for i in range(nbuf):
            fire(i, i)

        @pl.loop(0, n_ch, step=nbuf)
        def _(j):
            for i in range(nbuf):
                drain(j + i, i)

                @pl.when(j + i + nbuf < n_ch)
                def _():
                    fire(j + i + nbuf, i)

        for i in range(nbuf):
            pltpu.make_async_copy(
                wb[i], o_h.at[pl.ds(base + (n_ch - nbuf + i) * ch, ch)],
                wsem[i]).wait()

    return k(ts, tr, sidx, ridx)


def _sc_scatter(ne, ridx, zrows, np_rows):
    ep, d = ne.shape
    nw = _NC * _NS
    per_w = ep // nw
    n_ch = per_w // _CH
    rpt = np_rows // _NS

    @functools.partial(
        pl.kernel,
        out_type=jax.ShapeDtypeStruct((2 * np_rows, d), _F32),
        mesh=_sc_mesh(),
        scratch_types=[
            pltpu.VMEM((n_ch, _CH), jnp.int32),
            pltpu.VMEM((_CH, d), _F32),
            pltpu.VMEM((_CH, d), _F32),
            pltpu.VMEM_SHARED((np_rows, d), _F32),
            pltpu.SemaphoreType.DMA,
            pltpu.SemaphoreType.DMA,
        ],
    )
    def k(ne_h, ri_h, z_h, out_h, iv, b0, b1, acc, sem0, sem1):
        cid = lax.axis_index("c")
        sid = lax.axis_index("s")
        wid = sid * _NC + cid
        base = wid * per_w
        pltpu.sync_copy(z_h.at[pl.ds(sid * rpt, rpt)],
                        acc.at[pl.ds(sid * rpt, rpt)])
        pltpu.sync_copy(ri_h.at[wid], iv)
        plsc.subcore_barrier()

        def fire(j, b, sem):
            pltpu.async_copy(ne_h.at[pl.ds(base + j * _CH, _CH)], b, sem)

        def drain(j, b, sem):
            pltpu.make_async_copy(ne_h.at[pl.ds(base + j * _CH, _CH)], b, sem).wait()
            pltpu.sync_copy(b, acc.at[iv.at[j]], add=True)

        fire(0, b0, sem0)

        @pl.loop(0, n_ch, step=2)
        def _(j):
            fire(j + 1, b1, sem1)
            drain(j, b0, sem0)

            @pl.when(j + 2 < n_ch)
            def _():
                fire(j + 2, b0, sem0)

            drain(j + 1, b1, sem1)

        plsc.subcore_barrier()
        pltpu.sync_copy(acc.at[pl.ds(sid * rpt, rpt)],
                        out_h.at[pl.ds(cid * np_rows + sid * rpt, rpt)])

    return k(ne, ridx, zrows)


def kernel(nodes, edges, senders, receivers, globals_, params):
    n_nodes, d = nodes.shape
    n_edges = edges.shape[0]
    nw = _NC * _NS
    span = nw * _CH
    nsp = 4
    ep = -(-n_edges // (nsp * span)) * (nsp * span)
    epk = ep // nsp
    nb = 1024
    eb = 1280
    np_rows = -(-(n_nodes + 1) // nb) * nb
    assert np_rows % _NS == 0 and epk % eb == 0 and (epk // span) % 2 == 0

    nodes_p = jnp.pad(nodes, ((0, np_rows - n_nodes), (0, 0)))
    pad_idx = (n_nodes + jnp.arange(ep - n_edges, dtype=jnp.int32)
               % (np_rows - n_nodes)).astype(jnp.int32)
    idx_s = jnp.concatenate([senders, pad_idx])
    idx_r = jnp.concatenate([receivers, pad_idx])
    sndh = [idx_s[h * epk:(h + 1) * epk].reshape(nw, -1, _CH // 2)
            for h in range(nsp)]
    rcvh = [idx_r[h * epk:(h + 1) * epk].reshape(nw, -1, _CH // 2)
            for h in range(nsp)]
    rcvh_s = [idx_r[h * epk:(h + 1) * epk].reshape(nw, -1, _CH)
              for h in range(nsp)]
    g = globals_.reshape(1, -1).astype(_F32)
    zrows = jnp.zeros((np_rows, d), _F32)

    def w_sr_of(s):
        w1e = params['step_%d' % s]['edge_mlp'][0][0]
        return jnp.concatenate([w1e[d:2 * d], w1e[2 * d:3 * d]], axis=1)

    (w1, b1), (w2, b2) = params['embed_node']
    n, ps, pr = _tc_embed_node_proj(nodes_p, w1, b1, w2, b2, w_sr_of(0), nb)
    (w1, b1), (w2, b2) = params['embed_edge']
    eh = [_tc_mlp2(edges, w1, b1, w2, b2, eb, out_rows=epk,
                   block_off=h * (epk // eb), out_dtype=jnp.bfloat16)
          for h in range(nsp)]

    wg = jnp.concatenate(
        [w for s in range(3)
         for w in (params['step_%d' % s]['edge_mlp'][0][0][3 * d:],
                   params['step_%d' % s]['node_mlp'][0][0][2 * d:])], axis=1)
    bcat = jnp.concatenate(
        [b.reshape(1, -1) for s in range(3)
         for b in (params['step_%d' % s]['edge_mlp'][0][1],
                   params['step_%d' % s]['node_mlp'][0][1])], axis=1)
    brows = _tc_biasrows(g, wg, bcat).reshape(6, d)

    for s in range(3):
        p = params['step_%d' % s]
        w1e = p['edge_mlp'][0][0]
        w2e, b2e = p['edge_mlp'][1]
        w1n = p['node_mlp'][0][0]
        w2n, b2n = p['node_mlp'][1]
        lng = p['ln_g'].reshape(1, -1)
        lnb = p['ln_b'].reshape(1, -1)

        geh = [_sc_gather(ps, pr, sndh[h], rcvh[h], epk) for h in range(nsp)]
        recvs, e_next = [], []
        for h in range(nsp):
            ne_h, eo_h = _tc_edge(eh[h], geh[h], w1e[:d],
                                  brows[2 * s:2 * s + 1], w2e, b2e,
                                  lng, lnb, eb)
            e_next.append(eo_h)
            recvs.append(_sc_scatter(ne_h, rcvh_s[h], zrows, np_rows))
        eh = e_next
        out = _tc_node(n, recvs, w1n[:d], w1n[d:2 * d],
                       brows[2 * s + 1:2 * s + 2], w2n, b2n, lng, lnb, nb,
                       w_sr=None if s == 2 else w_sr_of(s + 1))
        if s == 2:
            n = out
        else:
            n, ps, pr = out

    (w1, b1), (w2, b2) = params['decode_node']
    out = _tc_mlp2(n, w1, b1, w2, b2, nb)
    return out[:n_nodes]


# speedup vs baseline: 1.0661x; 1.0661x over previous
import functools

import jax
import jax.numpy as jnp
from jax import lax
from jax.experimental import pallas as pl
from jax.experimental.pallas import tpu as pltpu
from jax.experimental.pallas import tpu_sc as plsc

_F32 = jnp.float32
_NC = 2
_NS = 16
_CH = 128


def _full(shape):
    return pl.BlockSpec(shape, lambda i: (0, 0))


def _rows(block_rows, width):
    return pl.BlockSpec((block_rows, width), lambda i: (i, 0))


_PARAMS = pltpu.CompilerParams(dimension_semantics=("arbitrary",))


def _tc_mlp2(x, w1, b1, w2, b2, block_rows, out_rows=None, block_off=0,
             out_dtype=None):
    rows, k = x.shape
    h = w1.shape[1]
    o = w2.shape[1]
    out_rows = rows if out_rows is None else out_rows
    max_blk = rows // block_rows - 1
    out_dtype = _F32 if out_dtype is None else out_dtype

    def body(x_ref, w1_ref, b1_ref, w2_ref, b2_ref, o_ref):
        t = jnp.dot(x_ref[...], w1_ref[...], preferred_element_type=_F32)
        t = jax.nn.gelu(t + b1_ref[...])
        y = jnp.dot(t, w2_ref[...], preferred_element_type=_F32) + b2_ref[...]
        o_ref[...] = y.astype(out_dtype)

    return pl.pallas_call(
        body,
        grid=(out_rows // block_rows,),
        in_specs=[pl.BlockSpec((block_rows, k),
                               lambda i: (jnp.minimum(i + block_off, max_blk), 0)),
                  _full((k, h)), _full((1, h)), _full((h, o)), _full((1, o))],
        out_specs=_rows(block_rows, o),
        out_shape=jax.ShapeDtypeStruct((out_rows, o), out_dtype),
        compiler_params=_PARAMS,
    )(x, w1, b1.reshape(1, -1), w2, b2.reshape(1, -1))


def _tc_embed_node_proj(x, w1, b1, w2, b2, w_sr, block_rows):
    rows, k = x.shape
    d = w2.shape[1]

    def body(x_ref, w1_ref, b1_ref, w2_ref, b2_ref, wsr_ref,
             o_ref, ps_ref, pr_ref):
        t = jnp.dot(x_ref[...], w1_ref[...], preferred_element_type=_F32)
        t = jax.nn.gelu(t + b1_ref[...])
        no = jnp.dot(t, w2_ref[...], preferred_element_type=_F32) + b2_ref[...]
        o_ref[...] = no
        tp = jnp.dot(no, wsr_ref[...], preferred_element_type=_F32)
        ps_ref[...] = tp[:, :d]
        pr_ref[...] = tp[:, d:]

    return pl.pallas_call(
        body,
        grid=(rows // block_rows,),
        in_specs=[_rows(block_rows, k), _full((k, d)), _full((1, d)),
                  _full((d, d)), _full((1, d)), _full((d, 2 * d))],
        out_specs=[_rows(block_rows, d)] * 3,
        out_shape=[jax.ShapeDtypeStruct((rows, d), _F32)] * 3,
        compiler_params=_PARAMS,
    )(x, w1, b1.reshape(1, -1), w2, b2.reshape(1, -1), w_sr)


def _tc_biasrows(g, wg, bcat):
    def body(g_ref, w_ref, b_ref, o_ref):
        o_ref[...] = (jnp.dot(g_ref[...], w_ref[...], preferred_element_type=_F32)
                      + b_ref[...])

    return pl.pallas_call(
        body,
        out_shape=jax.ShapeDtypeStruct(bcat.shape, _F32),
    )(g, wg, bcat)


def _layer_norm(z, g_row, b_row):
    m = jnp.mean(z, axis=-1, keepdims=True)
    v = jnp.mean((z - m) ** 2, axis=-1, keepdims=True)
    return (z - m) / jnp.sqrt(v + 1e-6) * g_row + b_row


def _tc_edge(e, ge, w1e, brow, w2, b2, lng, lnb, block_rows):
    rows, d = e.shape

    def body(e_ref, ge_ref, w1_ref, brow_ref, w2_ref, b2_ref,
             g_ref, b_ref, ne_ref, eo_ref):
        t = jnp.dot(e_ref[...], w1_ref[...], preferred_element_type=_F32)
        t = jax.nn.gelu(t + ge_ref[...] + brow_ref[...])
        ne = jnp.dot(t.astype(jnp.bfloat16), w2_ref[...],
                     preferred_element_type=_F32) + b2_ref[...]
        ne_ref[...] = ne
        z = e_ref[...].astype(_F32) + ne
        eo_ref[...] = _layer_norm(z, g_ref[...], b_ref[...]).astype(jnp.bfloat16)

    return pl.pallas_call(
        body,
        grid=(rows // block_rows,),
        in_specs=[_rows(block_rows, d), _rows(block_rows, d),
                  _full((d, d)), _full((1, d)), _full((d, d)), _full((1, d)),
                  _full((1, d)), _full((1, d))],
        out_specs=[_rows(block_rows, d), _rows(block_rows, d)],
        out_shape=[jax.ShapeDtypeStruct((rows, d), _F32),
                   jax.ShapeDtypeStruct((rows, d), jnp.bfloat16)],
        compiler_params=_PARAMS,
    )(e, ge, w1e.astype(jnp.bfloat16), brow, w2.astype(jnp.bfloat16),
      b2.reshape(1, -1), lng, lnb)


def _tc_node(n, recvs, wn, wr, brow, w2, b2, lng, lnb, block_rows,
             w_sr=None):
    rows, d = n.shape
    nblk = rows // block_rows
    nr = len(recvs)

    def compute(n_ref, r_refs, wn_ref, wr_ref, brow_ref, w2_ref, b2_ref,
                g_ref, b_ref):
        r = r_refs[0][...] + r_refs[1][...]
        for rr in r_refs[2:]:
            r = r + rr[...]
        t = (jnp.dot(n_ref[...], wn_ref[...], preferred_element_type=_F32)
             + jnp.dot(r, wr_ref[...], preferred_element_type=_F32))
        t = jax.nn.gelu(t + brow_ref[...])
        nn = jnp.dot(t, w2_ref[...], preferred_element_type=_F32) + b2_ref[...]
        return _layer_norm(n_ref[...] + nn, g_ref[...], b_ref[...])

    lo = pl.BlockSpec((block_rows, d), lambda i: (i, 0))
    hi = pl.BlockSpec((block_rows, d), lambda i: (i + nblk, 0))
    in_specs = ([lo] + [lo, hi] * nr
                + [_full((d, d)), _full((d, d)), _full((1, d)), _full((d, d)),
                   _full((1, d)), _full((1, d)), _full((1, d))])
    args = ([n] + [rv for rv in recvs for _ in range(2)]
            + [wn, wr, brow, w2, b2.reshape(1, -1), lng, lnb])

    if w_sr is None:
        def body(*refs):
            n_ref = refs[0]
            r_refs = refs[1:1 + 2 * nr]
            (wn_ref, wr_ref, brow_ref, w2_ref, b2_ref, g_ref, b_ref,
             no_ref) = refs[1 + 2 * nr:]
            no_ref[...] = compute(n_ref, r_refs, wn_ref, wr_ref, brow_ref,
                                  w2_ref, b2_ref, g_ref, b_ref)

        return pl.pallas_call(
            body,
            grid=(nblk,),
            in_specs=in_specs,
            out_specs=_rows(block_rows, d),
            out_shape=jax.ShapeDtypeStruct((rows, d), _F32),
            compiler_params=_PARAMS,
        )(*args)

    def body3(*refs):
        n_ref = refs[0]
        r_refs = refs[1:1 + 2 * nr]
        (wn_ref, wr_ref, brow_ref, w2_ref, b2_ref, g_ref, b_ref, wsr_ref,
         no_ref, ps_ref, pr_ref) = refs[1 + 2 * nr:]
        no = compute(n_ref, r_refs, wn_ref, wr_ref, brow_ref,
                     w2_ref, b2_ref, g_ref, b_ref)
        no_ref[...] = no
        tp = jnp.dot(no, wsr_ref[...], preferred_element_type=_F32)
        ps_ref[...] = tp[:, :d]
        pr_ref[...] = tp[:, d:]

    return pl.pallas_call(
        body3,
        grid=(nblk,),
        in_specs=in_specs + [_full((d, 2 * d))],
        out_specs=[_rows(block_rows, d)] * 3,
        out_shape=[jax.ShapeDtypeStruct((rows, d), _F32)] * 3,
        compiler_params=_PARAMS,
    )(*args, w_sr)


def _sc_mesh():
    return plsc.VectorSubcoreMesh(core_axis_name="c", subcore_axis_name="s")


def _sc_gather(ts, tr, sidx, ridx, ep):
    d = ts.shape[1]
    nv = d // 16
    nw = _NC * _NS
    per_w = ep // nw
    ch = _CH // 2
    n_ch = per_w // ch
    nbuf = 4

    @functools.partial(
        pl.kernel,
        out_type=jax.ShapeDtypeStruct((ep, d), _F32),
        mesh=_sc_mesh(),
        scratch_types=(
            [pltpu.VMEM((n_ch, ch), jnp.int32)] * 2
            + [pltpu.VMEM((ch, d), _F32)] * (3 * nbuf)
            + [pltpu.SemaphoreType.DMA] * (2 * nbuf)
        ),
    )
    def k(ts_h, tr_h, si_h, ri_h, o_h, iv_s, iv_r, *bufs):
        bs = bufs[0:nbuf]
        br = bufs[nbuf:2 * nbuf]
        wb = bufs[2 * nbuf:3 * nbuf]
        sem = bufs[3 * nbuf:4 * nbuf]
        wsem = bufs[4 * nbuf:5 * nbuf]
        wid = lax.axis_index("s") * _NC + lax.axis_index("c")
        base = wid * per_w
        pltpu.sync_copy(si_h.at[wid], iv_s)
        pltpu.sync_copy(ri_h.at[wid], iv_r)

        def fire(j, i):
            pltpu.async_copy(ts_h.at[iv_s.at[j]], bs[i], sem[i])
            pltpu.async_copy(tr_h.at[iv_r.at[j]], br[i], sem[i])

        def drain(j, i):
            pltpu.make_async_copy(ts_h.at[iv_s.at[j]], bs[i], sem[i]).wait()
            pltpu.make_async_copy(tr_h.at[iv_r.at[j]], br[i], sem[i]).wait()

            @pl.when(j >= nbuf)
            def _():
                pltpu.make_async_copy(
                    wb[i], o_h.at[pl.ds(base + (j - nbuf) * ch, ch)],
                    wsem[i]).wait()

            @pl.loop(0, ch, unroll=4)
            def _(r):
                for c in range(nv):
                    sl = pl.ds(c * 16, 16)
                    wb[i][r, sl] = bs[i][r, sl] + br[i][r, sl]

            pltpu.async_copy(wb[i], o_h.at[pl.ds(base + j * ch, ch)], wsem[i])

        for i in range(nbuf):
            fire(i, i)

        @pl.loop(0, n_ch, step=nbuf)
        def _(j):
            for i in range(nbuf):
                drain(j + i, i)

                @pl.when(j + i + nbuf < n_ch)
                def _():
                    fire(j + i + nbuf, i)

        for i in range(nbuf):
            pltpu.make_async_copy(
                wb[i], o_h.at[pl.ds(base + (n_ch - nbuf + i) * ch, ch)],
                wsem[i]).wait()

    return k(ts, tr, sidx, ridx)


def _sc_scatter(ne, ridx, zrows, np_rows):
    ep, d = ne.shape
    nw = _NC * _NS
    per_w = ep // nw
    n_ch = per_w // _CH
    rpt = np_rows // _NS

    @functools.partial(
        pl.kernel,
        out_type=jax.ShapeDtypeStruct((2 * np_rows, d), _F32),
        mesh=_sc_mesh(),
        scratch_types=[
            pltpu.VMEM((n_ch, _CH), jnp.int32),
            pltpu.VMEM((_CH, d), _F32),
            pltpu.VMEM((_CH, d), _F32),
            pltpu.VMEM_SHARED((np_rows, d), _F32),
            pltpu.SemaphoreType.DMA,
            pltpu.SemaphoreType.DMA,
        ],
    )
    def k(ne_h, ri_h, z_h, out_h, iv, b0, b1, acc, sem0, sem1):
        cid = lax.axis_index("c")
        sid = lax.axis_index("s")
        wid = sid * _NC + cid
        base = wid * per_w
        pltpu.sync_copy(z_h.at[pl.ds(sid * rpt, rpt)],
                        acc.at[pl.ds(sid * rpt, rpt)])
        pltpu.sync_copy(ri_h.at[wid], iv)
        plsc.subcore_barrier()

        def fire(j, b, sem):
            pltpu.async_copy(ne_h.at[pl.ds(base + j * _CH, _CH)], b, sem)

        def drain(j, b, sem):
            pltpu.make_async_copy(ne_h.at[pl.ds(base + j * _CH, _CH)], b, sem).wait()
            pltpu.sync_copy(b, acc.at[iv.at[j]], add=True)

        fire(0, b0, sem0)

        @pl.loop(0, n_ch, step=2)
        def _(j):
            fire(j + 1, b1, sem1)
            drain(j, b0, sem0)

            @pl.when(j + 2 < n_ch)
            def _():
                fire(j + 2, b0, sem0)

            drain(j + 1, b1, sem1)

        plsc.subcore_barrier()
        pltpu.sync_copy(acc.at[pl.ds(sid * rpt, rpt)],
                        out_h.at[pl.ds(cid * np_rows + sid * rpt, rpt)])

    return k(ne, ridx, zrows)


def kernel(nodes, edges, senders, receivers, globals_, params):
    n_nodes, d = nodes.shape
    n_edges = edges.shape[0]
    nw = _NC * _NS
    span = nw * _CH
    nsp = 2
    ep = -(-n_edges // (nsp * span)) * (nsp * span)
    epk = ep // nsp
    nb = 1024
    eb = 1280
    np_rows = -(-(n_nodes + 1) // nb) * nb
    assert np_rows % _NS == 0 and epk % eb == 0 and (epk // span) % 2 == 0

    nodes_p = jnp.pad(nodes, ((0, np_rows - n_nodes), (0, 0)))
    pad_idx = (n_nodes + jnp.arange(ep - n_edges, dtype=jnp.int32)
               % (np_rows - n_nodes)).astype(jnp.int32)
    idx_s = jnp.concatenate([senders, pad_idx])
    idx_r = jnp.concatenate([receivers, pad_idx])
    sndh = [idx_s[h * epk:(h + 1) * epk].reshape(nw, -1, _CH // 2)
            for h in range(nsp)]
    rcvh = [idx_r[h * epk:(h + 1) * epk].reshape(nw, -1, _CH // 2)
            for h in range(nsp)]
    rcvh_s = [idx_r[h * epk:(h + 1) * epk].reshape(nw, -1, _CH)
              for h in range(nsp)]
    g = globals_.reshape(1, -1).astype(_F32)
    zrows = jnp.zeros((np_rows, d), _F32)

    def w_sr_of(s):
        w1e = params['step_%d' % s]['edge_mlp'][0][0]
        return jnp.concatenate([w1e[d:2 * d], w1e[2 * d:3 * d]], axis=1)

    (w1, b1), (w2, b2) = params['embed_node']
    n, ps, pr = _tc_embed_node_proj(nodes_p, w1, b1, w2, b2, w_sr_of(0), nb)
    (w1, b1), (w2, b2) = params['embed_edge']
    eh = [_tc_mlp2(edges, w1, b1, w2, b2, eb, out_rows=epk,
                   block_off=h * (epk // eb), out_dtype=jnp.bfloat16)
          for h in range(nsp)]

    wg = jnp.concatenate(
        [w for s in range(3)
         for w in (params['step_%d' % s]['edge_mlp'][0][0][3 * d:],
                   params['step_%d' % s]['node_mlp'][0][0][2 * d:])], axis=1)
    bcat = jnp.concatenate(
        [b.reshape(1, -1) for s in range(3)
         for b in (params['step_%d' % s]['edge_mlp'][0][1],
                   params['step_%d' % s]['node_mlp'][0][1])], axis=1)
    brows = _tc_biasrows(g, wg, bcat).reshape(6, d)

    for s in range(3):
        p = params['step_%d' % s]
        w1e = p['edge_mlp'][0][0]
        w2e, b2e = p['edge_mlp'][1]
        w1n = p['node_mlp'][0][0]
        w2n, b2n = p['node_mlp'][1]
        lng = p['ln_g'].reshape(1, -1)
        lnb = p['ln_b'].reshape(1, -1)

        geh = [_sc_gather(ps, pr, sndh[h], rcvh[h], epk) for h in range(nsp)]
        recvs, e_next = [], []
        for h in range(nsp):
            ne_h, eo_h = _tc_edge(eh[h], geh[h], w1e[:d],
                                  brows[2 * s:2 * s + 1], w2e, b2e,
                                  lng, lnb, eb)
            e_next.append(eo_h)
            recvs.append(_sc_scatter(ne_h, rcvh_s[h], zrows, np_rows))
        eh = e_next
        out = _tc_node(n, recvs, w1n[:d], w1n[d:2 * d],
                       brows[2 * s + 1:2 * s + 2], w2n, b2n, lng, lnb, nb,
                       w_sr=None if s == 2 else w_sr_of(s + 1))
        if s == 2:
            n = out
        else:
            n, ps, pr = out

    (w1, b1), (w2, b2) = params['decode_node']
    out = _tc_mlp2(n, w1, b1, w2, b2, nb)
    return out[:n_nodes]


# speedup vs baseline: 1.1591x; 1.0873x over previous
import functools

import jax
import jax.numpy as jnp
from jax import lax
from jax.experimental import pallas as pl
from jax.experimental.pallas import tpu as pltpu
from jax.experimental.pallas import tpu_sc as plsc

_F32 = jnp.float32
_NC = 2
_NS = 16
_CH = 128


def _full(shape):
    return pl.BlockSpec(shape, lambda i: (0, 0))


def _rows(block_rows, width):
    return pl.BlockSpec((block_rows, width), lambda i: (i, 0))


_PARAMS = pltpu.CompilerParams(dimension_semantics=("arbitrary",))


def _tc_mlp2(x, w1, b1, w2, b2, block_rows, out_rows=None, block_off=0,
             out_dtype=None):
    rows, k = x.shape
    h = w1.shape[1]
    o = w2.shape[1]
    out_rows = rows if out_rows is None else out_rows
    max_blk = rows // block_rows - 1
    out_dtype = _F32 if out_dtype is None else out_dtype

    def body(x_ref, w1_ref, b1_ref, w2_ref, b2_ref, o_ref):
        t = jnp.dot(x_ref[...], w1_ref[...], preferred_element_type=_F32)
        t = jax.nn.gelu(t + b1_ref[...])
        y = jnp.dot(t, w2_ref[...], preferred_element_type=_F32) + b2_ref[...]
        o_ref[...] = y.astype(out_dtype)

    return pl.pallas_call(
        body,
        grid=(out_rows // block_rows,),
        in_specs=[pl.BlockSpec((block_rows, k),
                               lambda i: (jnp.minimum(i + block_off, max_blk), 0)),
                  _full((k, h)), _full((1, h)), _full((h, o)), _full((1, o))],
        out_specs=_rows(block_rows, o),
        out_shape=jax.ShapeDtypeStruct((out_rows, o), out_dtype),
        compiler_params=_PARAMS,
    )(x, w1, b1.reshape(1, -1), w2, b2.reshape(1, -1))


def _tc_embed_node_proj(x, w1, b1, w2, b2, w_sr, block_rows):
    rows, k = x.shape
    d = w2.shape[1]

    def body(x_ref, w1_ref, b1_ref, w2_ref, b2_ref, wsr_ref,
             o_ref, ps_ref, pr_ref):
        t = jnp.dot(x_ref[...], w1_ref[...], preferred_element_type=_F32)
        t = jax.nn.gelu(t + b1_ref[...])
        no = jnp.dot(t, w2_ref[...], preferred_element_type=_F32) + b2_ref[...]
        o_ref[...] = no
        tp = jnp.dot(no, wsr_ref[...], preferred_element_type=_F32)
        ps_ref[...] = tp[:, :d]
        pr_ref[...] = tp[:, d:]

    return pl.pallas_call(
        body,
        grid=(rows // block_rows,),
        in_specs=[_rows(block_rows, k), _full((k, d)), _full((1, d)),
                  _full((d, d)), _full((1, d)), _full((d, 2 * d))],
        out_specs=[_rows(block_rows, d)] * 3,
        out_shape=[jax.ShapeDtypeStruct((rows, d), _F32)] * 3,
        compiler_params=_PARAMS,
    )(x, w1, b1.reshape(1, -1), w2, b2.reshape(1, -1), w_sr)


def _tc_biasrows(g, wg, bcat):
    def body(g_ref, w_ref, b_ref, o_ref):
        o_ref[...] = (jnp.dot(g_ref[...], w_ref[...], preferred_element_type=_F32)
                      + b_ref[...])

    return pl.pallas_call(
        body,
        out_shape=jax.ShapeDtypeStruct(bcat.shape, _F32),
    )(g, wg, bcat)


def _layer_norm(z, g_row, b_row):
    m = jnp.mean(z, axis=-1, keepdims=True)
    v = jnp.mean((z - m) ** 2, axis=-1, keepdims=True)
    return (z - m) / jnp.sqrt(v + 1e-6) * g_row + b_row


def _tc_edge(e, ge, w1e, brow, w2, b2, lng, lnb, block_rows):
    rows, d = e.shape

    def body(e_ref, ge_ref, w1_ref, brow_ref, w2_ref, b2_ref,
             g_ref, b_ref, ne_ref, eo_ref):
        t = jnp.dot(e_ref[...], w1_ref[...], preferred_element_type=_F32)
        t = jax.nn.gelu(t + ge_ref[...] + brow_ref[...])
        ne = jnp.dot(t.astype(jnp.bfloat16), w2_ref[...],
                     preferred_element_type=_F32) + b2_ref[...]
        ne_ref[...] = ne
        z = e_ref[...].astype(_F32) + ne
        eo_ref[...] = _layer_norm(z, g_ref[...], b_ref[...]).astype(jnp.bfloat16)

    return pl.pallas_call(
        body,
        grid=(rows // block_rows,),
        in_specs=[_rows(block_rows, d), _rows(block_rows, d),
                  _full((d, d)), _full((1, d)), _full((d, d)), _full((1, d)),
                  _full((1, d)), _full((1, d))],
        out_specs=[_rows(block_rows, d), _rows(block_rows, d)],
        out_shape=[jax.ShapeDtypeStruct((rows, d), _F32),
                   jax.ShapeDtypeStruct((rows, d), jnp.bfloat16)],
        compiler_params=_PARAMS,
    )(e, ge, w1e.astype(jnp.bfloat16), brow, w2.astype(jnp.bfloat16),
      b2.reshape(1, -1), lng, lnb)


def _tc_node(n, recvs, wn, wr, brow, w2, b2, lng, lnb, block_rows,
             w_sr=None):
    rows, d = n.shape
    nblk = rows // block_rows
    nr = len(recvs)

    def compute(n_ref, r_refs, wn_ref, wr_ref, brow_ref, w2_ref, b2_ref,
                g_ref, b_ref):
        r = r_refs[0][...] + r_refs[1][...]
        for rr in r_refs[2:]:
            r = r + rr[...]
        t = (jnp.dot(n_ref[...], wn_ref[...], preferred_element_type=_F32)
             + jnp.dot(r, wr_ref[...], preferred_element_type=_F32))
        t = jax.nn.gelu(t + brow_ref[...])
        nn = jnp.dot(t, w2_ref[...], preferred_element_type=_F32) + b2_ref[...]
        return _layer_norm(n_ref[...] + nn, g_ref[...], b_ref[...])

    lo = pl.BlockSpec((block_rows, d), lambda i: (i, 0))
    hi = pl.BlockSpec((block_rows, d), lambda i: (i + nblk, 0))
    in_specs = ([lo] + [lo, hi] * nr
                + [_full((d, d)), _full((d, d)), _full((1, d)), _full((d, d)),
                   _full((1, d)), _full((1, d)), _full((1, d))])
    args = ([n] + [rv for rv in recvs for _ in range(2)]
            + [wn, wr, brow, w2, b2.reshape(1, -1), lng, lnb])

    if w_sr is None:
        def body(*refs):
            n_ref = refs[0]
            r_refs = refs[1:1 + 2 * nr]
            (wn_ref, wr_ref, brow_ref, w2_ref, b2_ref, g_ref, b_ref,
             no_ref) = refs[1 + 2 * nr:]
            no_ref[...] = compute(n_ref, r_refs, wn_ref, wr_ref, brow_ref,
                                  w2_ref, b2_ref, g_ref, b_ref)

        return pl.pallas_call(
            body,
            grid=(nblk,),
            in_specs=in_specs,
            out_specs=_rows(block_rows, d),
            out_shape=jax.ShapeDtypeStruct((rows, d), _F32),
            compiler_params=_PARAMS,
        )(*args)

    def body3(*refs):
        n_ref = refs[0]
        r_refs = refs[1:1 + 2 * nr]
        (wn_ref, wr_ref, brow_ref, w2_ref, b2_ref, g_ref, b_ref, wsr_ref,
         no_ref, ps_ref, pr_ref) = refs[1 + 2 * nr:]
        no = compute(n_ref, r_refs, wn_ref, wr_ref, brow_ref,
                     w2_ref, b2_ref, g_ref, b_ref)
        no_ref[...] = no
        tp = jnp.dot(no, wsr_ref[...], preferred_element_type=_F32)
        ps_ref[...] = tp[:, :d]
        pr_ref[...] = tp[:, d:]

    return pl.pallas_call(
        body3,
        grid=(nblk,),
        in_specs=in_specs + [_full((d, 2 * d))],
        out_specs=[_rows(block_rows, d)] * 3,
        out_shape=[jax.ShapeDtypeStruct((rows, d), _F32)] * 3,
        compiler_params=_PARAMS,
    )(*args, w_sr)


def _sc_mesh():
    return plsc.VectorSubcoreMesh(core_axis_name="c", subcore_axis_name="s")


def _sc_gather(ts, tr, sidx, ridx, ep):
    d = ts.shape[1]
    nv = d // 16
    nw = _NC * _NS
    per_w = ep // nw
    ch = _CH // 2
    n_ch = per_w // ch
    nbuf = 4

    @functools.partial(
        pl.kernel,
        out_type=jax.ShapeDtypeStruct((ep, d), _F32),
        mesh=_sc_mesh(),
        scratch_types=(
            [pltpu.VMEM((n_ch, ch), jnp.int32)] * 2
            + [pltpu.VMEM((ch, d), _F32)] * (3 * nbuf)
            + [pltpu.SemaphoreType.DMA] * (2 * nbuf)
        ),
    )
    def k(ts_h, tr_h, si_h, ri_h, o_h, iv_s, iv_r, *bufs):
        bs = bufs[0:nbuf]
        br = bufs[nbuf:2 * nbuf]
        wb = bufs[2 * nbuf:3 * nbuf]
        sem = bufs[3 * nbuf:4 * nbuf]
        wsem = bufs[4 * nbuf:5 * nbuf]
        wid = lax.axis_index("s") * _NC + lax.axis_index("c")
        base = wid * per_w
        pltpu.sync_copy(si_h.at[wid], iv_s)
        pltpu.sync_copy(ri_h.at[wid], iv_r)

        def fire(j, i):
            pltpu.async_copy(ts_h.at[iv_s.at[j]], bs[i], sem[i])
            pltpu.async_copy(tr_h.at[iv_r.at[j]], br[i], sem[i])

        def drain(j, i):
            pltpu.make_async_copy(ts_h.at[iv_s.at[j]], bs[i], sem[i]).wait()
            pltpu.make_async_copy(tr_h.at[iv_r.at[j]], br[i], sem[i]).wait()

            @pl.when(j >= nbuf)
            def _():
                pltpu.make_async_copy(
                    wb[i], o_h.at[pl.ds(base + (j - nbuf) * ch, ch)],
                    wsem[i]).wait()

            @pl.loop(0, ch, unroll=4)
            def _(r):
                for c in range(nv):
                    sl = pl.ds(c * 16, 16)
                    wb[i][r, sl] = bs[i][r, sl] + br[i][r, sl]

            pltpu.async_copy(wb[i], o_h.at[pl.ds(base + j * ch, ch)], wsem[i])

        for i in range(nbuf):
            fire(i, i)

        @pl.loop(0, n_ch, step=nbuf)
        def _(j):
            for i in range(nbuf):
                drain(j + i, i)

                @pl.when(j + i + nbuf < n_ch)
                def _():
                    fire(j + i + nbuf, i)

        for i in range(nbuf):
            pltpu.make_async_copy(
                wb[i], o_h.at[pl.ds(base + (n_ch - nbuf + i) * ch, ch)],
                wsem[i]).wait()

    return k(ts, tr, sidx, ridx)


def _sc_scatter(ne, ridx, zrows, np_rows):
    ep, d = ne.shape
    nw = _NC * _NS
    per_w = ep // nw
    n_ch = per_w // _CH
    rpt = np_rows // _NS

    @functools.partial(
        pl.kernel,
        out_type=jax.ShapeDtypeStruct((2 * np_rows, d), _F32),
        mesh=_sc_mesh(),
        scratch_types=[
            pltpu.VMEM((n_ch, _CH), jnp.int32),
            pltpu.VMEM((_CH, d), _F32),
            pltpu.VMEM((_CH, d), _F32),
            pltpu.VMEM_SHARED((np_rows, d), _F32),
            pltpu.SemaphoreType.DMA,
            pltpu.SemaphoreType.DMA,
        ],
    )
    def k(ne_h, ri_h, z_h, out_h, iv, b0, b1, acc, sem0, sem1):
        cid = lax.axis_index("c")
        sid = lax.axis_index("s")
        wid = sid * _NC + cid
        base = wid * per_w
        pltpu.sync_copy(z_h.at[pl.ds(sid * rpt, rpt)],
                        acc.at[pl.ds(sid * rpt, rpt)])
        pltpu.sync_copy(ri_h.at[wid], iv)
        plsc.subcore_barrier()

        def fire(j, b, sem):
            pltpu.async_copy(ne_h.at[pl.ds(base + j * _CH, _CH)], b, sem)

        def drain(j, b, sem):
            pltpu.make_async_copy(ne_h.at[pl.ds(base + j * _CH, _CH)], b, sem).wait()
            pltpu.sync_copy(b, acc.at[iv.at[j]], add=True)

        fire(0, b0, sem0)

        @pl.loop(0, n_ch, step=2)
        def _(j):
            fire(j + 1, b1, sem1)
            drain(j, b0, sem0)

            @pl.when(j + 2 < n_ch)
            def _():
                fire(j + 2, b0, sem0)

            drain(j + 1, b1, sem1)

        plsc.subcore_barrier()
        pltpu.sync_copy(acc.at[pl.ds(sid * rpt, rpt)],
                        out_h.at[pl.ds(cid * np_rows + sid * rpt, rpt)])

    return k(ne, ridx, zrows)


def kernel(nodes, edges, senders, receivers, globals_, params):
    n_nodes, d = nodes.shape
    n_edges = edges.shape[0]
    nw = _NC * _NS
    span = nw * _CH
    nsp = 2
    ep = -(-n_edges // (nsp * span)) * (nsp * span)
    epk = ep // nsp
    nb = 1024
    eb = 2560
    np_rows = -(-(n_nodes + 1) // nb) * nb
    assert np_rows % _NS == 0 and epk % eb == 0 and (epk // span) % 2 == 0

    nodes_p = jnp.pad(nodes, ((0, np_rows - n_nodes), (0, 0)))
    pad_idx = (n_nodes + jnp.arange(ep - n_edges, dtype=jnp.int32)
               % (np_rows - n_nodes)).astype(jnp.int32)
    idx_s = jnp.concatenate([senders, pad_idx])
    idx_r = jnp.concatenate([receivers, pad_idx])
    sndh = [idx_s[h * epk:(h + 1) * epk].reshape(nw, -1, _CH // 2)
            for h in range(nsp)]
    rcvh = [idx_r[h * epk:(h + 1) * epk].reshape(nw, -1, _CH // 2)
            for h in range(nsp)]
    rcvh_s = [idx_r[h * epk:(h + 1) * epk].reshape(nw, -1, _CH)
              for h in range(nsp)]
    g = globals_.reshape(1, -1).astype(_F32)
    zrows = jnp.zeros((np_rows, d), _F32)

    def w_sr_of(s):
        w1e = params['step_%d' % s]['edge_mlp'][0][0]
        return jnp.concatenate([w1e[d:2 * d], w1e[2 * d:3 * d]], axis=1)

    (w1, b1), (w2, b2) = params['embed_node']
    n, ps, pr = _tc_embed_node_proj(nodes_p, w1, b1, w2, b2, w_sr_of(0), nb)
    (w1, b1), (w2, b2) = params['embed_edge']
    eh = [_tc_mlp2(edges, w1, b1, w2, b2, eb, out_rows=epk,
                   block_off=h * (epk // eb), out_dtype=jnp.bfloat16)
          for h in range(nsp)]

    wg = jnp.concatenate(
        [w for s in range(3)
         for w in (params['step_%d' % s]['edge_mlp'][0][0][3 * d:],
                   params['step_%d' % s]['node_mlp'][0][0][2 * d:])], axis=1)
    bcat = jnp.concatenate(
        [b.reshape(1, -1) for s in range(3)
         for b in (params['step_%d' % s]['edge_mlp'][0][1],
                   params['step_%d' % s]['node_mlp'][0][1])], axis=1)
    brows = _tc_biasrows(g, wg, bcat).reshape(6, d)

    for s in range(3):
        p = params['step_%d' % s]
        w1e = p['edge_mlp'][0][0]
        w2e, b2e = p['edge_mlp'][1]
        w1n = p['node_mlp'][0][0]
        w2n, b2n = p['node_mlp'][1]
        lng = p['ln_g'].reshape(1, -1)
        lnb = p['ln_b'].reshape(1, -1)

        geh = [_sc_gather(ps, pr, sndh[h], rcvh[h], epk) for h in range(nsp)]
        recvs, e_next = [], []
        for h in range(nsp):
            ne_h, eo_h = _tc_edge(eh[h], geh[h], w1e[:d],
                                  brows[2 * s:2 * s + 1], w2e, b2e,
                                  lng, lnb, eb)
            e_next.append(eo_h)
            recvs.append(_sc_scatter(ne_h, rcvh_s[h], zrows, np_rows))
        eh = e_next
        out = _tc_node(n, recvs, w1n[:d], w1n[d:2 * d],
                       brows[2 * s + 1:2 * s + 2], w2n, b2n, lng, lnb, nb,
                       w_sr=None if s == 2 else w_sr_of(s + 1))
        if s == 2:
            n = out
        else:
            n, ps, pr = out

    (w1, b1), (w2, b2) = params['decode_node']
    out = _tc_mlp2(n, w1, b1, w2, b2, nb)
    return out[:n_nodes]
